# Initial kernel scaffold; baseline (speedup 1.0000x reference)
import jax
import jax.numpy as jnp
from jax.experimental import pallas as pl


def kernel(x, tables):
    raise NotImplementedError("write your pallas kernel here")


# speedup vs baseline: 64.4890x; 64.4890x over previous
import functools

import numpy as np
import jax
import jax.numpy as jnp
from jax import lax
from jax.experimental import pallas as pl
from jax.experimental.pallas import tpu as pltpu
from jax.experimental.pallas import tpu_sc as plsc

_FEAT = 2
_NL = 16
_MAX_RES, _MIN_RES = 2048, 16
_MAX_ENTRY = 2 ** 19
_MASK = _MAX_ENTRY - 1
_factor = np.exp((np.log(_MAX_RES) - np.log(_MIN_RES)) / (_NL - 1))
_RES = [float(np.floor(_MIN_RES * _factor ** i)) for i in range(_NL)]
_SIZES = [int(min(r ** 3, _MAX_ENTRY)) for r in _RES]
_ROW_OFF = [int(v) for v in np.cumsum([0] + _SIZES)[:-1]]
_TOTAL_ROWS = int(np.sum(_SIZES))
_PRIMES = (3367900313, 2654435761, 805459861)
_P32 = [int(p - 2 ** 32 if p >= 2 ** 31 else p) for p in _PRIMES]
_CORNERS = [(0, 0, 0), (0, 1, 0), (0, 0, 1), (0, 1, 1),
            (1, 0, 0), (1, 0, 1), (1, 1, 0), (1, 1, 1)]

_NC, _NS, _L = 2, 16, 16
_NW = _NC * _NS
_C = 256
_G = _C // _L
_B = 8 * _C
_B2 = 2 * _B
_NI = _B2 // 128

_pf = np.zeros((_NL, 2, _L), np.float32)
_pi = np.zeros((_NL, 4, _L), np.int32)
for _l in range(_NL):
    _r = _RES[_l]
    _pf[_l, 0] = np.float32(_r - 1)
    _pf[_l, 1] = np.float32(_r - 1.0001)
    _ri = int(_r)
    _pi[_l, 0] = _ri
    _pi[_l, 1] = _ri * _ri
    _pi[_l, 2] = 1 if _SIZES[_l] == _MAX_ENTRY else 0
    _pi[_l, 3] = _ROW_OFF[_l]
_PF = _pf.reshape(-1)
_PI = _pi.reshape(-1)


def _grid_body(npt, n_pts, xf_hbm, tab_hbm, pf_hbm, pi_hbm, out_hbm,
               x_v, pf_v, pi_v, idx_v, w_v, rows_v, out_v, sem):
    wid = lax.axis_index("s") * _NC + lax.axis_index("c")
    pt0 = wid * npt
    for d in range(3):
        pltpu.sync_copy(xf_hbm.at[pl.ds(d * n_pts + pt0, npt)],
                        x_v.at[pl.ds(d * npt, npt)])
    pltpu.sync_copy(pf_hbm, pf_v)
    pltpu.sync_copy(pi_hbm, pi_v)
    iota = lax.iota(jnp.int32, _L)
    n_steps = (npt // _C) * _NL

    def p_phase(t):
        chunk = t >> 4
        lvl = t & 15
        slot = t & 1
        pbase = chunk * _C
        fo = lvl * (2 * _L)
        res_m1 = pf_v[pl.ds(fo, _L)]
        clip_hi = pf_v[pl.ds(fo + _L, _L)]
        io = lvl * (4 * _L)
        r_v = pi_v[pl.ds(io, _L)]
        r2_v = pi_v[pl.ds(io + _L, _L)]
        hashed_v = pi_v[pl.ds(io + 2 * _L, _L)]
        off_v = pi_v[pl.ds(io + 3 * _L, _L)]
        hmask = hashed_v > 0
        slot_b = jnp.full((_L,), slot, jnp.int32)

        def g_body(g, carry):
            p0 = pbase + g * _L
            x0 = x_v[pl.ds(p0, _L)]
            x1 = x_v[pl.ds(npt + p0, _L)]
            x2 = x_v[pl.ds(2 * npt + p0, _L)]
            c0 = jnp.minimum(jnp.maximum(x0 * res_m1, 0.0), clip_hi)
            c1 = jnp.minimum(jnp.maximum(x1 * res_m1, 0.0), clip_hi)
            c2 = jnp.minimum(jnp.maximum(x2 * res_m1, 0.0), clip_hi)
            i0 = c0.astype(jnp.int32)
            i1 = c1.astype(jnp.int32)
            i2 = c2.astype(jnp.int32)
            d0 = c0 - i0.astype(jnp.float32)
            d1 = c1 - i1.astype(jnp.float32)
            d2 = c2 - i2.astype(jnp.float32)
            a0 = i0 * _P32[0]; a0b = a0 + _P32[0]
            a1 = i1 * _P32[1]; a1b = a1 + _P32[1]
            a2 = i2 * _P32[2]; a2b = a2 + _P32[2]
            b1 = i1 * r_v; b1b = b1 + r_v
            b2 = i2 * r2_v; b2b = b2 + r2_v
            i0p = i0 + 1
            mx = 1.0 - d0; my = 1.0 - d1; mz = 1.0 - d2
            wxy = (mx * my, d0 * my, mx * d1, d0 * d1)
            for j, (ox, oy, oz) in enumerate(_CORNERS):
                h = ((a0b if ox else a0) ^ (a1b if oy else a1) ^
                     (a2b if oz else a2)) & _MASK
                didx = ((i0p if ox else i0) + (b1b if oy else b1) +
                        (b2b if oz else b2))
                idx2 = (jnp.where(hmask, h, didx) + off_v) * 2
                fpos = j * _C + g * _L
                fpos2 = 2 * fpos
                row = jnp.full((_L,), fpos2 >> 7, jnp.int32)
                col2 = (fpos2 & 127) + 2 * iota
                plsc.store_scatter(idx_v, [slot_b, row, col2], idx2)
                plsc.store_scatter(idx_v, [slot_b, row, col2 + 1], idx2 + 1)
                wj = wxy[(1 if j & 1 else 0) + (2 if j & 2 else 0)]
                wj = wj * (d2 if j & 4 else mz)
                w_v[pl.ds(slot * _B + fpos, _L)] = wj
            return carry

        lax.fori_loop(jnp.int32(0), jnp.int32(_G), g_body, jnp.int32(0), unroll=False)

        def fire(i, carry):
            pltpu.async_copy(tab_hbm.at[idx_v.at[slot, i]],
                             rows_v.at[pl.ds(slot * _B2 + i * 128, 128)],
                             sem.at[slot])
            return carry

        lax.fori_loop(jnp.int32(0), jnp.int32(_NI), fire, jnp.int32(0), unroll=False)

    def a_phase(tp):
        chunkp = tp >> 4
        lvlp = tp & 15
        slotp = tp & 1
        cslot = chunkp & 1
        def drain(i, carry):
            pltpu.make_async_copy(tab_hbm.at[idx_v.at[slotp, i]],
                                  rows_v.at[pl.ds(slotp * _B2 + i * 128, 128)],
                                  sem.at[slotp]).wait()
            return carry

        lax.fori_loop(jnp.int32(0), jnp.int32(_NI), drain, jnp.int32(0), unroll=False)
        colbase = lvlp * 2

        def g_body(g, carry):
            obase = cslot * (_C * 32) + g * (_L * 32) + colbase
            acc0 = jnp.zeros((_L,), jnp.float32)
            acc1 = jnp.zeros((_L,), jnp.float32)
            for j in range(8):
                fpos = j * _C + g * _L
                wv = w_v[pl.ds(slotp * _B + fpos, _L)]
                rvec2 = slotp * _B2 + 2 * fpos + 2 * iota
                v0 = plsc.load_gather(rows_v, [rvec2])
                v1 = plsc.load_gather(rows_v, [rvec2 + 1])
                acc0 = acc0 + wv * v0
                acc1 = acc1 + wv * v1
            opos = obase + iota * 32
            plsc.store_scatter(out_v, [opos], acc0)
            plsc.store_scatter(out_v, [opos + 1], acc1)
            return carry

        lax.fori_loop(jnp.int32(0), jnp.int32(_G), g_body, jnp.int32(0), unroll=False)

        @pl.when(lvlp == jnp.int32(15))
        def _():
            gb = (pt0 + chunkp * _C) * 32
            pltpu.sync_copy(out_v.at[pl.ds(cslot * (_C * 32), _C * 32)],
                            out_hbm.at[pl.ds(gb, _C * 32)])

    def step(t, carry):
        @pl.when(t < jnp.int32(n_steps))
        def _():
            p_phase(t)

        @pl.when(t > jnp.int32(0))
        def _():
            a_phase(t - 1)

        return carry

    lax.fori_loop(jnp.int32(0), jnp.int32(n_steps + 1), step, jnp.int32(0), unroll=False)


def kernel(x, tables):
    n_pts = x.shape[0]
    assert n_pts % (_NW * _C) == 0
    npt = n_pts // _NW
    xf = jnp.transpose(x).reshape(-1)
    tab = jnp.concatenate(tables, axis=0).reshape(-1)
    mesh = plsc.VectorSubcoreMesh(core_axis_name="c", subcore_axis_name="s")
    kfn = pl.kernel(
        functools.partial(_grid_body, npt, n_pts),
        out_type=jax.ShapeDtypeStruct((n_pts * 2 * _NL,), jnp.float32),
        mesh=mesh,
        scratch_types=[
            pltpu.VMEM((3 * npt,), jnp.float32),
            pltpu.VMEM((_NL * 2 * _L,), jnp.float32),
            pltpu.VMEM((_NL * 4 * _L,), jnp.int32),
            pltpu.VMEM((2, _NI, 128), jnp.int32),
            pltpu.VMEM((2 * _B,), jnp.float32),
            pltpu.VMEM((2 * _B2,), jnp.float32),
            pltpu.VMEM((2 * _C * 32,), jnp.float32),
            pltpu.SemaphoreType.DMA((2,)),
        ],
        compiler_params=pltpu.CompilerParams(needs_layout_passes=False),
    )
    out = kfn(xf, tab, jnp.asarray(_PF), jnp.asarray(_PI))
    return out.reshape(n_pts, 2 * _NL)


# speedup vs baseline: 100.0014x; 1.5507x over previous
import functools

import numpy as np
import jax
import jax.numpy as jnp
from jax import lax
from jax.experimental import pallas as pl
from jax.experimental.pallas import tpu as pltpu
from jax.experimental.pallas import tpu_sc as plsc

_NL = 16
_MAX_RES, _MIN_RES = 2048, 16
_MAX_ENTRY = 2 ** 19
_MASK = _MAX_ENTRY - 1
_factor = np.exp((np.log(_MAX_RES) - np.log(_MIN_RES)) / (_NL - 1))
_RES = [float(np.floor(_MIN_RES * _factor ** i)) for i in range(_NL)]
_SIZES = [int(min(r ** 3, _MAX_ENTRY)) for r in _RES]
_HASHED = [s == _MAX_ENTRY for s in _SIZES]
_PRIMES = (3367900313, 2654435761, 805459861)
_P32 = [int(p - 2 ** 32 if p >= 2 ** 31 else p) for p in _PRIMES]
_CORNERS = [(0, 0, 0), (0, 1, 0), (0, 0, 1), (0, 1, 1),
            (1, 0, 0), (1, 0, 1), (1, 1, 0), (1, 1, 1)]

_NC, _NS, _L = 2, 16, 16
_NW = _NC * _NS
_C = 256
_G = _C // _L
_B = 8 * _C
_B2 = 2 * _B
_NI = _B2 // 128


def _grid_body(npt, n_pts, *refs):
    xf_hbm = refs[0]
    tabs = refs[1:1 + _NL]
    out_hbm = refs[1 + _NL]
    x_v, idx_v, w_v, rows_v, out_v, sem = refs[2 + _NL:]
    wid = lax.axis_index("s") * _NC + lax.axis_index("c")
    pt0 = wid * npt
    for d in range(3):
        pltpu.sync_copy(xf_hbm.at[pl.ds(d * n_pts + pt0, npt)],
                        x_v.at[pl.ds(d * npt, npt)])
    iota = lax.iota(jnp.int32, _L)
    n_chunks = npt // _C

    def p_phase(lvl, pbase):
        slot = lvl & 1
        res = _RES[lvl]
        res_m1 = np.float32(res - 1)
        clip_hi = np.float32(res - 1.0001)
        ri = int(res)
        ri2 = ri * ri
        hashed = _HASHED[lvl]

        def g_body(g, carry):
            p0 = pbase + g * _L
            x0 = x_v[pl.ds(p0, _L)]
            x1 = x_v[pl.ds(npt + p0, _L)]
            x2 = x_v[pl.ds(2 * npt + p0, _L)]
            c0 = jnp.minimum(jnp.maximum(x0 * res_m1, 0.0), clip_hi)
            c1 = jnp.minimum(jnp.maximum(x1 * res_m1, 0.0), clip_hi)
            c2 = jnp.minimum(jnp.maximum(x2 * res_m1, 0.0), clip_hi)
            i0 = c0.astype(jnp.int32)
            i1 = c1.astype(jnp.int32)
            i2 = c2.astype(jnp.int32)
            d0 = c0 - i0.astype(jnp.float32)
            d1 = c1 - i1.astype(jnp.float32)
            d2 = c2 - i2.astype(jnp.float32)
            if hashed:
                a0 = i0 * _P32[0]; a0b = a0 + _P32[0]
                a1 = i1 * _P32[1]; a1b = a1 + _P32[1]
                a2 = i2 * _P32[2]; a2b = a2 + _P32[2]
            else:
                b1 = i1 * ri; b1b = b1 + ri
                b2 = i2 * ri2; b2b = b2 + ri2
                i0p = i0 + 1
            mx = 1.0 - d0; my = 1.0 - d1; mz = 1.0 - d2
            wxy = (mx * my, d0 * my, mx * d1, d0 * d1)
            slot_b = jnp.full((_L,), slot, jnp.int32)
            for j, (ox, oy, oz) in enumerate(_CORNERS):
                if hashed:
                    idx = ((a0b if ox else a0) ^ (a1b if oy else a1) ^
                           (a2b if oz else a2)) & _MASK
                else:
                    idx = ((i0p if ox else i0) + (b1b if oy else b1) +
                           (b2b if oz else b2))
                idx2 = idx * 2
                fpos = j * _C + g * _L
                fpos2 = 2 * fpos
                row = jnp.full((_L,), fpos2 >> 7, jnp.int32)
                col2 = (fpos2 & 127) + 2 * iota
                plsc.store_scatter(idx_v, [slot_b, row, col2], idx2)
                plsc.store_scatter(idx_v, [slot_b, row, col2 + 1], idx2 + 1)
                wj = wxy[(1 if j & 1 else 0) + (2 if j & 2 else 0)]
                wj = wj * (d2 if j & 4 else mz)
                w_v[pl.ds(slot * _B + fpos, _L)] = wj
            return carry

        lax.fori_loop(jnp.int32(0), jnp.int32(_G), g_body, jnp.int32(0),
                      unroll=False)

        def fire(i, carry):
            pltpu.async_copy(tabs[lvl].at[idx_v.at[jnp.int32(slot), i]],
                             rows_v.at[pl.ds(slot * _B2 + i * 128, 128)],
                             sem.at[jnp.int32(slot)])
            return carry

        lax.fori_loop(jnp.int32(0), jnp.int32(_NI), fire, jnp.int32(0),
                      unroll=False)

    def a_phase(lvl, pbase):
        slot = lvl & 1

        def drain(i, carry):
            pltpu.make_async_copy(
                tabs[lvl].at[idx_v.at[jnp.int32(slot), i]],
                rows_v.at[pl.ds(slot * _B2 + i * 128, 128)],
                sem.at[jnp.int32(slot)]).wait()
            return carry

        lax.fori_loop(jnp.int32(0), jnp.int32(_NI), drain, jnp.int32(0),
                      unroll=False)

        def g_body(g, carry):
            obase = g * (_L * 32) + lvl * 2
            acc0 = jnp.zeros((_L,), jnp.float32)
            acc1 = jnp.zeros((_L,), jnp.float32)
            for j in range(8):
                fpos = j * _C + g * _L
                wv = w_v[pl.ds(slot * _B + fpos, _L)]
                rvec2 = slot * _B2 + 2 * fpos + 2 * iota
                v0 = plsc.load_gather(rows_v, [rvec2])
                v1 = plsc.load_gather(rows_v, [rvec2 + 1])
                acc0 = acc0 + wv * v0
                acc1 = acc1 + wv * v1
            opos = obase + iota * 32
            plsc.store_scatter(out_v, [opos], acc0)
            plsc.store_scatter(out_v, [opos + 1], acc1)
            return carry

        lax.fori_loop(jnp.int32(0), jnp.int32(_G), g_body, jnp.int32(0),
                      unroll=False)

    def chunk_body(c, carry):
        pbase = c * _C
        p_phase(0, pbase)
        for lvl in range(1, _NL):
            p_phase(lvl, pbase)
            a_phase(lvl - 1, pbase)
        a_phase(_NL - 1, pbase)
        gb = (pt0 + pbase) * 32
        pltpu.sync_copy(out_v, out_hbm.at[pl.ds(gb, _C * 32)])
        return carry

    lax.fori_loop(jnp.int32(0), jnp.int32(n_chunks), chunk_body,
                  jnp.int32(0), unroll=False)


def kernel(x, tables):
    n_pts = x.shape[0]
    assert n_pts % (_NW * _C) == 0
    npt = n_pts // _NW
    xf = jnp.transpose(x).reshape(-1)
    flats = [t.reshape(-1) for t in tables]
    mesh = plsc.VectorSubcoreMesh(core_axis_name="c", subcore_axis_name="s")
    kfn = pl.kernel(
        functools.partial(_grid_body, npt, n_pts),
        out_type=jax.ShapeDtypeStruct((n_pts * 2 * _NL,), jnp.float32),
        mesh=mesh,
        scratch_types=[
            pltpu.VMEM((3 * npt,), jnp.float32),
            pltpu.VMEM((2, _NI, 128), jnp.int32),
            pltpu.VMEM((2 * _B,), jnp.float32),
            pltpu.VMEM((2 * _B2,), jnp.float32),
            pltpu.VMEM((_C * 32,), jnp.float32),
            pltpu.SemaphoreType.DMA((2,)),
        ],
        compiler_params=pltpu.CompilerParams(needs_layout_passes=False),
    )
    out = kfn(xf, *flats)
    return out.reshape(n_pts, 2 * _NL)
